# Initial kernel scaffold; baseline (speedup 1.0000x reference)
import jax
import jax.numpy as jnp
from jax.experimental import pallas as pl


def kernel(encoder_inputs, attention_weight):
    raise NotImplementedError("write your pallas kernel here")


# speedup vs baseline: 1.6386x; 1.6386x over previous
import functools

import jax
import jax.numpy as jnp
from jax import lax
from jax.experimental import pallas as pl
from jax.experimental.pallas import tpu as pltpu
from jax.experimental.pallas import tpu_sc as plsc

VOCAB = 100000
LANES = 16


def _build(batch, seq):
    groups = (seq + LANES - 1) // LANES
    seq_pad = groups * LANES
    mesh = plsc.VectorSubcoreMesh(core_axis_name="c", subcore_axis_name="s")
    info = plsc.get_sparse_core_info()
    num_workers = info.num_cores * info.num_subcores
    rows_per_w = batch // num_workers

    @functools.partial(
        pl.kernel,
        mesh=mesh,
        out_type=jax.ShapeDtypeStruct((batch, VOCAB), jnp.float32),
        scratch_types=[
            pltpu.VMEM((seq_pad,), jnp.int32),
            pltpu.VMEM((seq_pad,), jnp.float32),
            pltpu.VMEM((VOCAB,), jnp.float32),
            pltpu.VMEM((LANES,), jnp.int32),
            pltpu.VMEM((LANES,), jnp.float32),
        ],
        compiler_params=pltpu.CompilerParams(needs_layout_passes=False),
    )
    def pg_kernel(ids_hbm, w_hbm, out_hbm, idx_v, w_v, rowbuf, ibuf, fbuf):
        wid = lax.axis_index("s") * info.num_cores + lax.axis_index("c")
        izeros = jnp.zeros((LANES,), jnp.int32)
        fzeros = jnp.zeros((LANES,), jnp.float32)
        iota = lax.iota(jnp.int32, LANES)
        prev_idx = jnp.maximum(iota - 1, 0)
        next_idx = jnp.minimum(iota + 1, LANES - 1)

        for g in range(groups):
            idx_v[pl.ds(g * LANES, LANES)] = izeros
            w_v[pl.ds(g * LANES, LANES)] = fzeros

        def zero_body(i, carry):
            rowbuf[pl.ds(i * LANES, LANES)] = fzeros
            return carry

        lax.fori_loop(0, VOCAB // LANES, zero_body, 0)

        def row_body(r0, carry):
            r = wid * rows_per_w + r0
            pltpu.sync_copy(ids_hbm.at[pl.ds(r * seq, seq)],
                            idx_v.at[pl.ds(0, seq)])
            pltpu.sync_copy(w_hbm.at[pl.ds(r * seq, seq)],
                            w_v.at[pl.ds(0, seq)])
            for g in range(groups):
                kk = idx_v[pl.ds(g * LANES, LANES)]
                vv = w_v[pl.ds(g * LANES, LANES)]
                ks, vs = plsc.sort_key_val(kk, vv)
                ibuf[...] = ks
                prev = plsc.load_gather(ibuf, [prev_idx])
                knext = plsc.load_gather(ibuf, [next_idx])
                is_start = (iota == 0) | (ks != prev)
                is_end = (iota == LANES - 1) | (ks != knext)
                csum = plsc.cumsum(vs)
                fbuf[...] = csum
                startidx = plsc.cummax(jnp.where(is_start, iota, 0))
                cprev = plsc.load_gather(fbuf, [jnp.maximum(startidx - 1, 0)])
                seg = jnp.where(startidx == 0, csum, csum - cprev)
                plsc.addupdate_scatter(rowbuf, [ks], seg, mask=is_end)
            pltpu.sync_copy(rowbuf, out_hbm.at[r])
            for g in range(groups):
                kk = idx_v[pl.ds(g * LANES, LANES)]
                plsc.store_scatter(rowbuf, [kk], fzeros)
            return carry

        lax.fori_loop(0, rows_per_w, row_body, 0)

    return pg_kernel


def kernel(encoder_inputs, attention_weight):
    batch, seq = encoder_inputs.shape
    ids = encoder_inputs.astype(jnp.int32).reshape(-1)
    w = attention_weight.astype(jnp.float32).reshape(-1)
    return _build(batch, seq)(ids, w)
